# baseline (speedup 1.0000x reference)
import functools

import jax
import jax.numpy as jnp
from jax.experimental import pallas as pl
from jax.experimental.pallas import tpu as pltpu

_K = 5
_BIG_IDX = 2**30
_LANES = 128


def _top5_mask_exact(dist, iota):
    mask = jnp.zeros(dist.shape, dtype=jnp.bool_)
    for _ in range(_K):
        m = jnp.min(dist, axis=1, keepdims=True)
        idx = jnp.min(jnp.where(dist == m, iota, _BIG_IDX), axis=1, keepdims=True)
        sel = iota == idx
        mask = jnp.logical_or(mask, sel)
        dist = jnp.where(sel, jnp.inf, dist)
    return mask


def _strip_min3(dist):
    r, n = dist.shape
    v1 = jnp.full((r, _LANES), jnp.inf, jnp.float32)
    v2 = v1
    v3 = v1
    for s in range(n // _LANES):
        t = dist[:, s * _LANES:(s + 1) * _LANES]
        nv1 = jnp.minimum(v1, t)
        t2 = jnp.maximum(v1, t)
        nv2 = jnp.minimum(v2, t2)
        t3 = jnp.maximum(v2, t2)
        v3 = jnp.minimum(v3, t3)
        v1, v2 = nv1, nv2
    return jnp.concatenate([v1, v2, v3], axis=1)


def _fifth_smallest(cand):
    m = None
    for _ in range(_K):
        m = jnp.min(cand, axis=1, keepdims=True)
        cand = jnp.where(cand == m, jnp.inf, cand)
    return m


def _knn_overlap_kernel(xp_ref, xq_ref, tp_ref, tq_ref, acc_ref, *, rows):
    i = pl.program_id(0)
    n = xq_ref.shape[1]

    def dist_block(p_ref, q_ref):
        pr = p_ref[pl.ds(i * rows, rows), :]
        return jax.lax.dot(
            pr, q_ref[:, :], preferred_element_type=jnp.float32,
        )

    dist_x = dist_block(xp_ref, xq_ref)
    dist_t = dist_block(tp_ref, tq_ref)

    tx = _fifth_smallest(_strip_min3(dist_x))
    tt = _fifth_smallest(_strip_min3(dist_t))
    mx = jnp.where(dist_x <= tx, 1.0, 0.0)
    mt = jnp.where(dist_t <= tt, 1.0, 0.0)
    total = jnp.sum(mx) + jnp.sum(mt)

    acc_ref[0, 0, 0] = jnp.sum(mx * mt)

    @pl.when(total != jnp.float32(2 * _K * rows))
    def _exact_fallback():
        iota = jax.lax.broadcasted_iota(jnp.int32, (rows, n), 1)
        mx = _top5_mask_exact(dist_x, iota)
        mt = _top5_mask_exact(dist_t, iota)
        acc_ref[0, 0, 0] = jnp.sum(
            jnp.logical_and(mx, mt).astype(jnp.float32))


def _split_bf16(u):
    hi = u.astype(jnp.bfloat16)
    lo = (u - hi.astype(jnp.float32)).astype(jnp.bfloat16)
    return hi, lo


def _augment(x):
    sq = jnp.sum(x * x, axis=1, keepdims=True)
    sqh = sq.astype(jnp.bfloat16)
    sqm_f = sq - sqh.astype(jnp.float32)
    sqm = sqm_f.astype(jnp.bfloat16)
    sql = (sqm_f - sqm.astype(jnp.float32)).astype(jnp.bfloat16)
    uh, ul = _split_bf16(-2.0 * x)
    vh, vl = _split_bf16(x)
    ones = jnp.ones_like(sqh)
    p = jnp.concatenate([uh, uh, ul, ones, ones, ones], axis=1)
    q = jnp.concatenate([vh, vl, vh, sqh, sqm, sql], axis=1).T
    return p, q


@jax.jit
def kernel(input, target):
    n, d = input.shape
    rows = 256
    grid = (n // rows,)
    xp, xq = _augment(input)
    tp, tq = _augment(target)
    kdim = 3 * d + 3
    p_spec = pl.BlockSpec((n, kdim), lambda i: (0, 0))
    q_spec = pl.BlockSpec((kdim, n), lambda i: (0, 0))
    partial = pl.pallas_call(
        functools.partial(_knn_overlap_kernel, rows=rows),
        grid=grid,
        in_specs=[p_spec, q_spec, p_spec, q_spec],
        out_specs=pl.BlockSpec(
            (1, 1, 1), lambda i: (i, 0, 0), memory_space=pltpu.SMEM
        ),
        out_shape=jax.ShapeDtypeStruct((grid[0], 1, 1), jnp.float32),
        compiler_params=pltpu.CompilerParams(
            dimension_semantics=("arbitrary",)
        ),
    )(xp, xq, tp, tq)
    loss = 1.0 - jnp.sum(partial) / (n * _K)
    return loss.astype(jnp.float32)


# speedup vs baseline: 1.0459x; 1.0459x over previous
import functools

import jax
import jax.numpy as jnp
from jax.experimental import pallas as pl
from jax.experimental.pallas import tpu as pltpu

_K = 5
_BIG_IDX = 2**30
_LANES = 128


def _top5_mask_exact(dist, iota):
    mask = jnp.zeros(dist.shape, dtype=jnp.bool_)
    for _ in range(_K):
        m = jnp.min(dist, axis=1, keepdims=True)
        idx = jnp.min(jnp.where(dist == m, iota, _BIG_IDX), axis=1, keepdims=True)
        sel = iota == idx
        mask = jnp.logical_or(mask, sel)
        dist = jnp.where(sel, jnp.inf, dist)
    return mask


def _strip_min2(dist, width=2 * _LANES):
    r, n = dist.shape
    v1 = jnp.full((r, width), jnp.inf, jnp.float32)
    v2 = v1
    for s in range(n // width):
        t = dist[:, s * width:(s + 1) * width]
        nv1 = jnp.minimum(v1, t)
        t2 = jnp.maximum(v1, t)
        v2 = jnp.minimum(v2, t2)
        v1 = nv1
    return jnp.concatenate([v1, v2], axis=1)


def _strip_min3(dist):
    r, n = dist.shape
    v1 = jnp.full((r, _LANES), jnp.inf, jnp.float32)
    v2 = v1
    v3 = v1
    for s in range(n // _LANES):
        t = dist[:, s * _LANES:(s + 1) * _LANES]
        nv1 = jnp.minimum(v1, t)
        t2 = jnp.maximum(v1, t)
        nv2 = jnp.minimum(v2, t2)
        t3 = jnp.maximum(v2, t2)
        v3 = jnp.minimum(v3, t3)
        v1, v2 = nv1, nv2
    return jnp.concatenate([v1, v2, v3], axis=1)


def _fifth_smallest(cand):
    m = None
    for _ in range(_K):
        m = jnp.min(cand, axis=1, keepdims=True)
        cand = jnp.where(cand == m, jnp.inf, cand)
    return m


def _knn_overlap_kernel(xp_ref, xq_ref, tp_ref, tq_ref, acc_ref, *, rows):
    i = pl.program_id(0)
    n = xq_ref.shape[1]

    def dist_block(p_ref, q_ref):
        pr = p_ref[pl.ds(i * rows, rows), :]
        return jax.lax.dot(
            pr, q_ref[:, :], preferred_element_type=jnp.float32,
        )

    dist_x = dist_block(xp_ref, xq_ref)
    dist_t = dist_block(tp_ref, tq_ref)

    full = jnp.float32(2 * _K * rows)

    def masks_and_total(tx, tt):
        mx = jnp.where(dist_x <= tx, 1.0, 0.0)
        mt = jnp.where(dist_t <= tt, 1.0, 0.0)
        return mx, mt, jnp.sum(mx + mt)

    mx, mt, total = masks_and_total(
        _fifth_smallest(_strip_min2(dist_x)),
        _fifth_smallest(_strip_min2(dist_t)))
    acc_ref[0, 0, 0] = jnp.sum(mx * mt)

    @pl.when(total != full)
    def _tier2():
        mx2, mt2, total2 = masks_and_total(
            _fifth_smallest(_strip_min3(dist_x)),
            _fifth_smallest(_strip_min3(dist_t)))
        acc_ref[0, 0, 0] = jnp.sum(mx2 * mt2)

        @pl.when(total2 != full)
        def _exact_fallback():
            iota = jax.lax.broadcasted_iota(jnp.int32, (rows, n), 1)
            ex = _top5_mask_exact(dist_x, iota)
            et = _top5_mask_exact(dist_t, iota)
            acc_ref[0, 0, 0] = jnp.sum(
                jnp.logical_and(ex, et).astype(jnp.float32))


def _split_bf16(u):
    hi = u.astype(jnp.bfloat16)
    lo = (u - hi.astype(jnp.float32)).astype(jnp.bfloat16)
    return hi, lo


def _augment(x):
    sq = jnp.sum(x * x, axis=1, keepdims=True)
    sqh = sq.astype(jnp.bfloat16)
    sqm_f = sq - sqh.astype(jnp.float32)
    sqm = sqm_f.astype(jnp.bfloat16)
    sql = (sqm_f - sqm.astype(jnp.float32)).astype(jnp.bfloat16)
    uh, ul = _split_bf16(-2.0 * x)
    vh, vl = _split_bf16(x)
    ones = jnp.ones_like(sqh)
    p = jnp.concatenate([uh, uh, ul, ones, ones, ones], axis=1)
    q = jnp.concatenate([vh, vl, vh, sqh, sqm, sql], axis=1).T
    return p, q


@jax.jit
def kernel(input, target):
    n, d = input.shape
    rows = 128
    grid = (n // rows,)
    xp, xq = _augment(input)
    tp, tq = _augment(target)
    kdim = 3 * d + 3
    p_spec = pl.BlockSpec((n, kdim), lambda i: (0, 0))
    q_spec = pl.BlockSpec((kdim, n), lambda i: (0, 0))
    partial = pl.pallas_call(
        functools.partial(_knn_overlap_kernel, rows=rows),
        grid=grid,
        in_specs=[p_spec, q_spec, p_spec, q_spec],
        out_specs=pl.BlockSpec(
            (1, 1, 1), lambda i: (i, 0, 0), memory_space=pltpu.SMEM
        ),
        out_shape=jax.ShapeDtypeStruct((grid[0], 1, 1), jnp.float32),
        compiler_params=pltpu.CompilerParams(
            dimension_semantics=("arbitrary",)
        ),
    )(xp, xq, tp, tq)
    loss = 1.0 - jnp.sum(partial) / (n * _K)
    return loss.astype(jnp.float32)
